# speedup vs baseline: 2.2036x; 2.2036x over previous
import functools

import jax
import jax.numpy as jnp
from jax import lax
from jax.experimental import pallas as pl
from jax.experimental.pallas import tpu as pltpu
from jax.experimental.pallas import tpu_sc as plsc

VOCAB = 100000
D_W = 128
D_P = 32
D_OUT = D_W + D_P
B = 1024
S = 200
N = B * S

NC = 2
NS = 16
NW = NC * NS
ROWS_PER_W = N // NW
CHUNK = 128
STEPS = ROWS_PER_W // CHUNK

_mesh = plsc.VectorSubcoreMesh(core_axis_name="c", subcore_axis_name="s")


@functools.partial(
    pl.kernel,
    mesh=_mesh,
    out_type=jax.ShapeDtypeStruct((N, D_OUT), jnp.float32),
    scratch_types=[
        pltpu.VMEM((CHUNK,), jnp.int32),
        pltpu.VMEM((CHUNK, D_W), jnp.float32),
        pltpu.VMEM((CHUNK, D_P), jnp.float32),
        pltpu.SemaphoreType.DMA,
    ],
)
def _emb_kernel(tok_hbm, pos_hbm, w_hbm, out_hbm, idx_v, rows_v, pos_v, sem):
    wid = lax.axis_index("s") * NC + lax.axis_index("c")
    base = wid * ROWS_PER_W

    def step(i, carry):
        r0 = base + i * CHUNK
        pltpu.sync_copy(tok_hbm.at[pl.ds(r0, CHUNK)], idx_v)
        pltpu.async_copy(w_hbm.at[idx_v], rows_v, sem).wait()
        pltpu.sync_copy(pos_hbm.at[pl.ds(r0, CHUNK)], pos_v)
        pltpu.sync_copy(rows_v, out_hbm.at[pl.ds(r0, CHUNK), pl.ds(0, D_W)])
        pltpu.sync_copy(pos_v, out_hbm.at[pl.ds(r0, CHUNK), pl.ds(D_W, D_P)])
        return carry

    lax.fori_loop(0, STEPS, step, 0)


def kernel(token_ids, pos_onehot, W):
    tok = token_ids.reshape(N).astype(jnp.int32)
    pos = pos_onehot.reshape(N, D_P)
    out = _emb_kernel(tok, pos, W)
    return out.reshape(B, S, D_OUT)
